# Initial kernel scaffold; baseline (speedup 1.0000x reference)
import jax
import jax.numpy as jnp
from jax.experimental import pallas as pl


def kernel(h, edge_index, theta0_w, theta0_b, phi0_w, phi0_b, theta1_w, theta1_b, phi1_w, phi1_b, cls_w, cls_b):
    raise NotImplementedError("write your pallas kernel here")


# speedup vs baseline: 1.5041x; 1.5041x over previous
import functools

import jax
import jax.numpy as jnp
from jax import lax
from jax.experimental import pallas as pl
from jax.experimental.pallas import tpu as pltpu
from jax.experimental.pallas import tpu_sc as plsc

N = 10000
E = 320000
D = 128
H = 128
C = 10

NW = 32
NPW = 320
N_PAD = NW * NPW
TRASH = NPW
ACC_ROWS = NPW + 8
NEG = -3.0e38
NEG_THRESH = -1.0e38
CE = 6400
NCHUNK = E // CE
SUBG = CE // 16
BATCH = 128


_SC_MESH = plsc.VectorSubcoreMesh(core_axis_name="c", subcore_axis_name="s")


@functools.partial(
    pl.kernel,
    mesh=_SC_MESH,
    out_type=jax.ShapeDtypeStruct((N_PAD, H), jnp.float32),
    compiler_params=pltpu.CompilerParams(needs_layout_passes=False),
    scratch_types=[
        pltpu.VMEM((ACC_ROWS, H), jnp.float32),
        pltpu.VMEM((CE,), jnp.int32),
        pltpu.VMEM((CE,), jnp.int32),
        pltpu.VMEM((BATCH,), jnp.int32),
        pltpu.VMEM((BATCH,), jnp.int32),
        pltpu.VMEM((BATCH, H), jnp.float32),
        pltpu.SemaphoreType.DMA,
    ],
)
def _segmax(a_hbm, src_hbm, dst_hbm, out_hbm,
            acc, srcc, dstc, sbuf, dbuf, rows, sem):
    wid = lax.axis_index("s") * 2 + lax.axis_index("c")
    lo = wid * NPW
    neg16 = jnp.full((16,), NEG, jnp.float32)
    zero16 = jnp.zeros((16,), jnp.int32)
    trash16 = jnp.full((16,), TRASH, jnp.int32)

    def _init_acc(i, carry):
        for k in range(H // 16):
            acc[i, pl.ds(16 * k, 16)] = neg16
        return carry

    lax.fori_loop(0, ACC_ROWS, _init_acc, 0)
    for j in range(BATCH // 16):
        sbuf[pl.ds(16 * j, 16)] = zero16
        dbuf[pl.ds(16 * j, 16)] = trash16

    def _flush(cur):
        for j in range(BATCH // 16):
            lane = lax.iota(jnp.int32, 16) + 16 * j
            keep = lane < cur
            dseg = dbuf[pl.ds(16 * j, 16)]
            sseg = sbuf[pl.ds(16 * j, 16)]
            dbuf[pl.ds(16 * j, 16)] = jnp.where(keep, dseg, trash16)
            sbuf[pl.ds(16 * j, 16)] = jnp.where(keep, sseg, zero16)
        pltpu.async_copy(a_hbm.at[sbuf], rows, sem).wait()

        def _accum(t, carry):
            dvec = dbuf[pl.ds(t * 16, 16)]
            for l in range(16):
                dl = dvec[l]
                e = t * 16 + l
                for k in range(H // 16):
                    sl = pl.ds(16 * k, 16)
                    acc[dl, sl] = jnp.maximum(acc[dl, sl], rows[e, sl])
            return carry

        lax.fori_loop(0, BATCH // 16, _accum, 0)

    def _chunk(c, cur):
        pltpu.sync_copy(src_hbm.at[pl.ds(c * CE, CE)], srcc)
        pltpu.sync_copy(dst_hbm.at[pl.ds(c * CE, CE)], dstc)

        def _sub(g, cur):
            full = cur > BATCH - 16

            @pl.when(full)
            def _():
                _flush(cur)

            cur = jnp.where(full, 0, cur)
            d = dstc[pl.ds(g * 16, 16)]
            s = srcc[pl.ds(g * 16, 16)]
            m = (d >= lo) & (d < lo + NPW)
            csum = plsc.cumsum(m.astype(jnp.int32))
            pos = csum + (cur - 1)
            plsc.store_scatter(sbuf, [pos], s, mask=m)
            plsc.store_scatter(dbuf, [pos], d - lo, mask=m)
            return cur + jnp.max(csum)

        return lax.fori_loop(0, SUBG, _sub, cur)

    cur = lax.fori_loop(0, NCHUNK, _chunk, 0)

    @pl.when(cur > 0)
    def _():
        _flush(cur)

    pltpu.sync_copy(acc.at[pl.ds(0, NPW)], out_hbm.at[pl.ds(lo, NPW)])


_DOT11 = (((1,), (1,)), ((), ()))
_BR = 1024


def _dense1_body(x_ref, tw_ref, pw_ref, tb_ref, pb_ref, a_ref, b_ref):
    x = x_ref[...]
    a = lax.dot_general(x, tw_ref[...], _DOT11, preferred_element_type=jnp.float32)
    p = lax.dot_general(x, pw_ref[...], _DOT11, preferred_element_type=jnp.float32)
    a_ref[...] = a
    b_ref[...] = p - a + tb_ref[...] + pb_ref[...]


def _dense2_body(m_ref, bp_ref, tw_ref, pw_ref, tb_ref, pb_ref, a_ref, b_ref):
    m = m_ref[...]
    x = jnp.where(m > NEG_THRESH, m + bp_ref[...], 0.0)
    a = lax.dot_general(x, tw_ref[...], _DOT11, preferred_element_type=jnp.float32)
    p = lax.dot_general(x, pw_ref[...], _DOT11, preferred_element_type=jnp.float32)
    a_ref[...] = a
    b_ref[...] = p - a + tb_ref[...] + pb_ref[...]


def _row_spec():
    return pl.BlockSpec((_BR, H), lambda i: (i, 0))


def _full_spec(shape):
    return pl.BlockSpec(shape, lambda i: tuple(0 for _ in shape))


def _dense1(x, tw, pw, tb, pb):
    return pl.pallas_call(
        _dense1_body,
        grid=(N_PAD // _BR,),
        in_specs=[_row_spec(), _full_spec((H, D)), _full_spec((H, D)),
                  _full_spec((1, H)), _full_spec((1, H))],
        out_specs=[_row_spec(), _row_spec()],
        out_shape=[jax.ShapeDtypeStruct((N_PAD, H), jnp.float32)] * 2,
    )(x, tw, pw, tb, pb)


def _dense2(m, bp, tw, pw, tb, pb):
    return pl.pallas_call(
        _dense2_body,
        grid=(N_PAD // _BR,),
        in_specs=[_row_spec(), _row_spec(), _full_spec((H, H)), _full_spec((H, H)),
                  _full_spec((1, H)), _full_spec((1, H))],
        out_specs=[_row_spec(), _row_spec()],
        out_shape=[jax.ShapeDtypeStruct((N_PAD, H), jnp.float32)] * 2,
    )(m, bp, tw, pw, tb, pb)


def _readout_body(m_ref, b_ref, cw_ref, cb_ref, o_ref, acc_ref):
    i = pl.program_id(0)

    @pl.when(i == 0)
    def _():
        acc_ref[...] = jnp.zeros_like(acc_ref)

    m = m_ref[...]
    h2 = jnp.where(m > NEG_THRESH, m + b_ref[...], 0.0)
    acc_ref[0:1, :] += jnp.sum(h2, axis=0, keepdims=True)

    @pl.when(i == pl.num_programs(0) - 1)
    def _():
        s = acc_ref[0:1, :] * (1.0 / N)
        r = lax.dot_general(s, cw_ref[...], _DOT11,
                            preferred_element_type=jnp.float32) + cb_ref[...]
        o_ref[...] = jnp.zeros_like(o_ref)
        o_ref[0:1, 0:16] = r


def _readout(m, b, cwp, cbp):
    return pl.pallas_call(
        _readout_body,
        grid=(N_PAD // _BR,),
        in_specs=[_row_spec(), _row_spec(), _full_spec((16, H)),
                  _full_spec((1, 16))],
        out_specs=_full_spec((8, 128)),
        out_shape=jax.ShapeDtypeStruct((8, 128), jnp.float32),
        scratch_shapes=[pltpu.VMEM((8, H), jnp.float32)],
    )(m, b, cwp, cbp)


def kernel(h, edge_index, theta0_w, theta0_b, phi0_w, phi0_b,
           theta1_w, theta1_b, phi1_w, phi1_b, cls_w, cls_b):
    src = edge_index[0]
    dst = edge_index[1]
    h_pad = jnp.pad(h, ((0, N_PAD - N), (0, 0)))

    a0, b0 = _dense1(h_pad, theta0_w, phi0_w,
                     theta0_b.reshape(1, H), phi0_b.reshape(1, H))
    m0 = _segmax(a0, src, dst)
    a1, b1 = _dense2(m0, b0, theta1_w, phi1_w,
                     theta1_b.reshape(1, H), phi1_b.reshape(1, H))
    m1 = _segmax(a1, src, dst)

    cwp = jnp.pad(cls_w, ((0, 16 - C), (0, 0)))
    cbp = jnp.pad(cls_b.reshape(1, C), ((0, 0), (0, 16 - C)))
    out = _readout(m1, b1, cwp, cbp)
    return out[0:1, 0:C]
